# baseline (speedup 1.0000x reference)
import jax
import jax.numpy as jnp
from jax import lax
from jax.experimental import pallas as pl
from jax.experimental.pallas import tpu as pltpu
from jax.experimental.pallas import tpu_sc as plsc

N = 10000
D = 128
E = 320000
NC, NS = 2, 16
NW = NC * NS
C = 80
CHUNKS = 128
EPT = CHUNKS * C
E_PAD = NW * EPT
N_PAD = 10240
NPT = N_PAD // NS
ZCH = 80


def _sc_body(row_hbm, col_hbm, w_hbm, x_hbm, parts_hbm,
             acc, g0, g1, s0, s1,
             ri0, ri1, ri2, ri3, ci0, ci1, ci2, ci3, wi0, wi1, wi2, wi3,
             semg0, semg1, sems0, sems1, semi0, semi1, semi2, semi3):
    c = lax.axis_index("c")
    s = lax.axis_index("s")
    wid = c * NS + s
    gbuf = (g0, g1)
    sbuf = (s0, s1)
    rowi = (ri0, ri1, ri2, ri3)
    coli = (ci0, ci1, ci2, ci3)
    wvi = (wi0, wi1, wi2, wi3)
    semg = (semg0, semg1)
    sems = (sems0, sems1)
    semi = (semi0, semi1, semi2, semi3)

    zero16 = jnp.zeros((16,), jnp.float32)

    def zero_row(r, carry):
        for j in range(D // 16):
            g0[r, pl.ds(j * 16, 16)] = zero16
        return carry

    lax.fori_loop(0, ZCH, zero_row, 0)

    for k in range(NPT // ZCH):
        pltpu.sync_copy(g0.at[pl.ds(0, ZCH)],
                        acc.at[pl.ds(s * NPT + k * ZCH, ZCH)])
    plsc.subcore_barrier()

    def idx_fetch(i, q):
        pltpu.async_copy(row_hbm.at[wid, i], rowi[q], semi[q])
        pltpu.async_copy(col_hbm.at[wid, i], coli[q], semi[q])
        pltpu.async_copy(w_hbm.at[wid, i], wvi[q], semi[q])

    def idx_wait(i, q):
        pltpu.make_async_copy(row_hbm.at[wid, i], rowi[q], semi[q]).wait()
        pltpu.make_async_copy(col_hbm.at[wid, i], coli[q], semi[q]).wait()
        pltpu.make_async_copy(w_hbm.at[wid, i], wvi[q], semi[q]).wait()

    def gather_issue(b, q):
        pltpu.async_copy(x_hbm.at[coli[q]], gbuf[b], semg[b])

    def body(i, phase):
        b, q = phase % 2, phase
        pltpu.make_async_copy(x_hbm.at[coli[q]], gbuf[b], semg[b]).wait()

        @pl.when(i >= 2)
        def _():
            pltpu.make_async_copy(sbuf[b], acc.at[rowi[q]], sems[b]).wait()

        def scale_group(gidx, carry2):
            w16 = wvi[q][pl.ds(gidx * 16, 16)]
            for e in range(16):
                wb = lax.broadcast(w16[e], (16,))
                r = gidx * 16 + e
                for j in range(D // 16):
                    sl = pl.ds(j * 16, 16)
                    sbuf[b][r, sl] = gbuf[b][r, sl] * wb
            return carry2

        lax.fori_loop(0, C // 16, scale_group, 0)
        pltpu.async_copy(sbuf[b], acc.at[rowi[q]], sems[b], add=True)

        @pl.when(i + 1 <= CHUNKS - 1)
        def _():
            idx_wait(i + 1, (phase + 1) % 4)
            gather_issue((phase + 1) % 2, (phase + 1) % 4)

        @pl.when((i >= 2) & (i + 2 <= CHUNKS - 1))
        def _():
            idx_fetch(i + 2, (phase + 2) % 4)

    for i in range(4):
        idx_fetch(i, i)
    idx_wait(0, 0)
    gather_issue(0, 0)

    def outer(kk, carry):
        for u in range(4):
            body(kk * 4 + u, u)
        return carry

    lax.fori_loop(0, CHUNKS // 4, outer, 0)

    pltpu.make_async_copy(sbuf[0], acc.at[rowi[0]], sems[0]).wait()
    pltpu.make_async_copy(sbuf[1], acc.at[rowi[1]], sems[1]).wait()

    plsc.subcore_barrier()

    for k in range(NPT // ZCH):
        rb = s * NPT + k * ZCH
        pltpu.sync_copy(acc.at[pl.ds(rb, ZCH)], g0.at[pl.ds(0, ZCH)])
        pltpu.sync_copy(g0.at[pl.ds(0, ZCH)], parts_hbm.at[c, pl.ds(rb, ZCH)])


def _sc_scatter(row3, col3, w3, x):
    mesh = plsc.VectorSubcoreMesh(core_axis_name="c", subcore_axis_name="s",
                                  num_cores=NC, num_subcores=NS)
    return pl.kernel(
        _sc_body,
        out_type=jax.ShapeDtypeStruct((NC, N_PAD, D), jnp.float32),
        mesh=mesh,
        scratch_types=[
            pltpu.VMEM_SHARED((N_PAD, D), jnp.float32),
            pltpu.VMEM((C, D), jnp.float32),
            pltpu.VMEM((C, D), jnp.float32),
            pltpu.VMEM((C, D), jnp.float32),
            pltpu.VMEM((C, D), jnp.float32),
            pltpu.VMEM((C,), jnp.int32),
            pltpu.VMEM((C,), jnp.int32),
            pltpu.VMEM((C,), jnp.int32),
            pltpu.VMEM((C,), jnp.int32),
            pltpu.VMEM((C,), jnp.int32),
            pltpu.VMEM((C,), jnp.int32),
            pltpu.VMEM((C,), jnp.int32),
            pltpu.VMEM((C,), jnp.int32),
            pltpu.VMEM((C,), jnp.float32),
            pltpu.VMEM((C,), jnp.float32),
            pltpu.VMEM((C,), jnp.float32),
            pltpu.VMEM((C,), jnp.float32),
            pltpu.SemaphoreType.DMA,
            pltpu.SemaphoreType.DMA,
            pltpu.SemaphoreType.DMA,
            pltpu.SemaphoreType.DMA,
            pltpu.SemaphoreType.DMA,
            pltpu.SemaphoreType.DMA,
            pltpu.SemaphoreType.DMA,
            pltpu.SemaphoreType.DMA,
        ],
    )(row3, col3, w3, x)


def _combine_body(p_ref, d_ref, o_ref):
    o_ref[...] = (p_ref[0] + p_ref[1]) * d_ref[...]


def _combine(parts, deg2d):
    bn = 2000
    return pl.pallas_call(
        _combine_body,
        out_shape=jax.ShapeDtypeStruct((N, D), jnp.float32),
        grid=(N // bn,),
        in_specs=[
            pl.BlockSpec((NC, bn, D), lambda i: (0, i, 0)),
            pl.BlockSpec((bn, 1), lambda i: (i, 0)),
        ],
        out_specs=pl.BlockSpec((bn, D), lambda i: (i, 0)),
    )(parts, deg2d)


def kernel(x, edge_index, edge_weight, deg_inv):
    row = edge_index[0].astype(jnp.int32)
    col = edge_index[1].astype(jnp.int32)
    w = edge_weight.astype(jnp.float32)
    pad = E_PAD - E
    row3 = jnp.concatenate([row, jnp.zeros((pad,), jnp.int32)]).reshape(NW, CHUNKS, C)
    col3 = jnp.concatenate([col, jnp.zeros((pad,), jnp.int32)]).reshape(NW, CHUNKS, C)
    w3 = jnp.concatenate([w, jnp.zeros((pad,), jnp.float32)]).reshape(NW, CHUNKS, C)
    parts = _sc_scatter(row3, col3, w3, x)
    return _combine(parts, deg_inv[:, None])


# speedup vs baseline: 1.2328x; 1.2328x over previous
import jax
import jax.numpy as jnp
from jax import lax
from jax.experimental import pallas as pl
from jax.experimental.pallas import tpu as pltpu
from jax.experimental.pallas import tpu_sc as plsc

N = 10000
D = 128
E = 320000
NC, NS = 2, 16
NW = NC * NS
C = 88
CHUNKS = 116
EPT = CHUNKS * C
E_PAD = NW * EPT
N_PAD = 10112
NPT = N_PAD // NS
WCH = [C] * 7 + [16]


def _sc_body(aux_hbm, x_hbm, parts_hbm,
             acc, g0, g1, s0, s1, a0, a1, a2, a3,
             semg0, semg1, sems0, sems1, semi0, semi1, semi2, semi3):
    c = lax.axis_index("c")
    s = lax.axis_index("s")
    wid = c * NS + s
    gbuf = (g0, g1)
    sbuf = (s0, s1)
    aux = (a0, a1, a2, a3)
    semg = (semg0, semg1)
    sems = (sems0, sems1)
    semi = (semi0, semi1, semi2, semi3)

    zero16 = jnp.zeros((16,), jnp.float32)

    def zero_row(r, carry):
        for j in range(D // 16):
            g0[r, pl.ds(j * 16, 16)] = zero16
        return carry

    lax.fori_loop(0, C, zero_row, 0)

    off = 0
    for sz in WCH:
        pltpu.sync_copy(g0.at[pl.ds(0, sz)],
                        acc.at[pl.ds(s * NPT + off, sz)])
        off += sz
    plsc.subcore_barrier()

    def idx_fetch(i, q):
        pltpu.async_copy(aux_hbm.at[wid, i], aux[q], semi[q])

    def idx_wait(i, q):
        pltpu.make_async_copy(aux_hbm.at[wid, i], aux[q], semi[q]).wait()

    def gather_issue(b, q):
        pltpu.async_copy(x_hbm.at[aux[q].at[1]], gbuf[b], semg[b])

    def body(i, phase):
        b, q = phase % 2, phase
        b1, q1 = (phase + 1) % 2, (phase + 1) % 4
        pltpu.make_async_copy(x_hbm.at[aux[q].at[1]], gbuf[b], semg[b]).wait()

        @pl.when(i >= 2)
        def _():
            pltpu.make_async_copy(sbuf[b], acc.at[aux[q].at[0]], sems[b]).wait()

        @pl.when(i + 1 <= CHUNKS - 1)
        def _():
            idx_wait(i + 1, q1)
            gather_issue(b1, q1)

        @pl.when((i >= 2) & (i + 2 <= CHUNKS - 1))
        def _():
            idx_fetch(i + 2, (phase + 2) % 4)

        def scale_group(gidx, carry2):
            base = jnp.minimum(gidx * 16, C - 16)
            w16 = lax.bitcast_convert_type(aux[q][2, pl.ds(base, 16)],
                                           jnp.float32)
            for e in range(16):
                wb = lax.broadcast(w16[e], (16,))
                r = base + e
                for j in range(D // 16):
                    sl = pl.ds(j * 16, 16)
                    sbuf[b][r, sl] = gbuf[b][r, sl] * wb
            return carry2

        lax.fori_loop(0, (C + 15) // 16, scale_group, 0)
        pltpu.async_copy(sbuf[b], acc.at[aux[q].at[0]], sems[b], add=True)

    for i in range(4):
        idx_fetch(i, i)
    idx_wait(0, 0)
    gather_issue(0, 0)

    def outer(kk, carry):
        for u in range(4):
            body(kk * 4 + u, u)
        return carry

    lax.fori_loop(0, CHUNKS // 4, outer, 0)

    pltpu.make_async_copy(sbuf[0], acc.at[aux[0].at[0]], sems[0]).wait()
    pltpu.make_async_copy(sbuf[1], acc.at[aux[1].at[0]], sems[1]).wait()

    plsc.subcore_barrier()

    off = 0
    for sz in WCH:
        rb = s * NPT + off
        pltpu.sync_copy(acc.at[pl.ds(rb, sz)], g0.at[pl.ds(0, sz)])
        pltpu.sync_copy(g0.at[pl.ds(0, sz)], parts_hbm.at[c, pl.ds(rb, sz)])
        off += sz


def _sc_scatter(aux, x):
    mesh = plsc.VectorSubcoreMesh(core_axis_name="c", subcore_axis_name="s",
                                  num_cores=NC, num_subcores=NS)
    return pl.kernel(
        _sc_body,
        out_type=jax.ShapeDtypeStruct((NC, N_PAD, D), jnp.float32),
        mesh=mesh,
        scratch_types=[
            pltpu.VMEM_SHARED((N_PAD, D), jnp.float32),
            pltpu.VMEM((C, D), jnp.float32),
            pltpu.VMEM((C, D), jnp.float32),
            pltpu.VMEM((C, D), jnp.float32),
            pltpu.VMEM((C, D), jnp.float32),
            pltpu.VMEM((3, C), jnp.int32),
            pltpu.VMEM((3, C), jnp.int32),
            pltpu.VMEM((3, C), jnp.int32),
            pltpu.VMEM((3, C), jnp.int32),
            pltpu.SemaphoreType.DMA,
            pltpu.SemaphoreType.DMA,
            pltpu.SemaphoreType.DMA,
            pltpu.SemaphoreType.DMA,
            pltpu.SemaphoreType.DMA,
            pltpu.SemaphoreType.DMA,
            pltpu.SemaphoreType.DMA,
            pltpu.SemaphoreType.DMA,
        ],
    )(aux, x)


def _combine_body(p_ref, d_ref, o_ref):
    o_ref[...] = (p_ref[0] + p_ref[1]) * d_ref[...]


def _combine(parts, deg2d):
    bn = 2000
    return pl.pallas_call(
        _combine_body,
        out_shape=jax.ShapeDtypeStruct((N, D), jnp.float32),
        grid=(N // bn,),
        in_specs=[
            pl.BlockSpec((NC, bn, D), lambda i: (0, i, 0)),
            pl.BlockSpec((bn, 1), lambda i: (i, 0)),
        ],
        out_specs=pl.BlockSpec((bn, D), lambda i: (i, 0)),
    )(parts, deg2d)


def kernel(x, edge_index, edge_weight, deg_inv):
    row = edge_index[0].astype(jnp.int32)
    col = edge_index[1].astype(jnp.int32)
    wbits = lax.bitcast_convert_type(edge_weight.astype(jnp.float32), jnp.int32)
    pad = E_PAD - E
    zpad = jnp.zeros((pad,), jnp.int32)
    rowp = jnp.concatenate([row, zpad]).reshape(NW, CHUNKS, C)
    colp = jnp.concatenate([col, zpad]).reshape(NW, CHUNKS, C)
    wp = jnp.concatenate([wbits, zpad]).reshape(NW, CHUNKS, C)
    aux = jnp.stack([rowp, colp, wp], axis=2)
    parts = _sc_scatter(aux, x)
    return _combine(parts, deg_inv[:, None])
